# baseline (speedup 1.0000x reference)
import functools

import jax
import jax.numpy as jnp
from jax import lax
from jax.experimental import pallas as pl
from jax.experimental.pallas import tpu as pltpu
from jax.experimental.pallas import tpu_sc as plsc


def _dpdp_body(lam_ref, f_ref, c_ref, beta_ref, gamma_ref, pc_ref, a_vm):
    f = f_ref[:]
    c = c_ref[:]
    T = f.shape[0]
    K = c.shape[0]
    lam = lam_ref[0]

    fn2 = jnp.sum(f * f, axis=1, keepdims=True)
    cn2 = jnp.sum(c * c, axis=1, keepdims=True).T
    g = jax.lax.dot_general(
        f, c, (((1,), (1,)), ((), ())),
        preferred_element_type=jnp.float32,
        precision=jax.lax.Precision.HIGHEST,
    )
    d2 = jnp.maximum(fn2 + cn2 - 2.0 * g, 0.0)

    mu = jnp.sum(d2) / jnp.float32(T * K)
    x = d2 - mu
    sh = 1
    while sh < T:
        x = x + jnp.concatenate([jnp.zeros((sh, K), jnp.float32), x[:-sh]], axis=0)
        sh *= 2
    pc_ref[:] = x

    row1 = pc_ref[pl.ds(0, 1), :]
    a1 = jnp.min(row1) + lam
    a_vm[pl.ds(0, 1), :] = jnp.reshape(a1, (1, 1))

    def dp_step(t, carry):
        m, a_prev = carry
        pj = pc_ref[pl.ds(t - 2, 1), :]
        pt = pc_ref[pl.ds(t - 1, 1), :]
        m = jnp.minimum(m, a_prev - pj)
        a_t = jnp.min(pt + m) + lam
        a_vm[pl.ds(t - 1, 1), :] = jnp.reshape(a_t, (1, 1))
        return (m, a_t)

    m0 = jnp.zeros((1, K), jnp.float32)
    jax.lax.fori_loop(2, T + 1, dp_step, (m0, a1), unroll=False)

    pc = pc_ref[:]
    avec = a_vm[:]
    qfull = avec - pc
    q = jnp.concatenate([jnp.zeros((1, K), jnp.float32), qfull[:-1]], axis=0)
    mv = q
    ji = jax.lax.broadcasted_iota(jnp.int32, (T, K), 0)
    inf = jnp.float32(jnp.inf)
    sh = 1
    while sh < T:
        pv = jnp.concatenate([jnp.full((sh, K), inf, jnp.float32), mv[:-sh]], axis=0)
        pj = jnp.concatenate([jnp.zeros((sh, K), jnp.int32), ji[:-sh]], axis=0)
        take = pv <= mv
        mv = jnp.where(take, pv, mv)
        ji = jnp.where(take, pj, ji)
        sh *= 2
    r = pc + mv
    rmin = jnp.min(r, axis=1, keepdims=True)
    kiota = jax.lax.broadcasted_iota(jnp.int32, (1, K), 1)
    gam = jnp.min(jnp.where(r == rmin, kiota, K), axis=1, keepdims=True)
    bet = jnp.min(jnp.where(kiota == gam, ji, jnp.int32(2**30)),
                  axis=1, keepdims=True)
    beta_ref[:] = bet
    gamma_ref[:] = gam


def _dpdp_beta_gamma(features, codebook, lam_arr):
    T, _ = features.shape
    K = codebook.shape[0]
    return pl.pallas_call(
        _dpdp_body,
        out_shape=[
            jax.ShapeDtypeStruct((T, 1), jnp.int32),
            jax.ShapeDtypeStruct((T, 1), jnp.int32),
        ],
        in_specs=[
            pl.BlockSpec(memory_space=pltpu.SMEM),
            pl.BlockSpec(memory_space=pltpu.VMEM),
            pl.BlockSpec(memory_space=pltpu.VMEM),
        ],
        out_specs=[
            pl.BlockSpec(memory_space=pltpu.VMEM),
            pl.BlockSpec(memory_space=pltpu.VMEM),
        ],
        scratch_shapes=[
            pltpu.VMEM((T, K), jnp.float32),
            pltpu.VMEM((T, 1), jnp.float32),
        ],
    )(lam_arr, features, codebook)


def _sc_backtrace_gather(T, K, D):
    tpw = T // 16
    mesh = plsc.VectorSubcoreMesh(core_axis_name="c", subcore_axis_name="s")

    @functools.partial(
        pl.kernel,
        mesh=mesh,
        out_type=[
            jax.ShapeDtypeStruct((T,), jnp.int32),
            jax.ShapeDtypeStruct((T, D), jnp.float32),
        ],
        scratch_types=[
            pltpu.VMEM((T + 16,), jnp.int32),
            pltpu.VMEM((T + 16,), jnp.int32),
            pltpu.VMEM((T,), jnp.int32),
            pltpu.VMEM((tpw,), jnp.int32),
            pltpu.VMEM((tpw, D), jnp.float32),
            pltpu.SemaphoreType.DMA,
        ],
    )
    def sc_fn(beta_hbm, gamma_hbm, cb_hbm, units_hbm, quant_hbm,
              beta_v, gamma_v, units_v, idx_v, rows_v, sem):
        cid = lax.axis_index("c")
        sid = lax.axis_index("s")
        liota = lax.broadcasted_iota(jnp.int32, (16,), 0)

        @pl.when((cid == 0) & (sid == 0))
        def _backtrace():
            pltpu.sync_copy(beta_hbm, beta_v.at[pl.ds(0, T)])
            pltpu.sync_copy(gamma_hbm, gamma_v.at[pl.ds(0, T)])

            b0 = beta_v[pl.ds(T - 1, 16)][0]
            g0 = gamma_v[pl.ds(T - 1, 16)][0]

            def pos_body(i, carry):
                b, g = carry
                p = T - 1 - i
                start = pl.multiple_of((p // 16) * 16, 16)
                vec = units_v[pl.ds(start, 16)]
                mask = (start + liota) == p
                units_v[pl.ds(start, 16)] = jnp.where(mask, g, vec)
                hit = p == b
                nxt = jnp.maximum(b - 1, 0)
                nb = beta_v[pl.ds(nxt, 16)][0]
                ng = gamma_v[pl.ds(nxt, 16)][0]
                b = jnp.where(hit, nb, b)
                g = jnp.where(hit, ng, g)
                return (b, g)

            lax.fori_loop(0, T, pos_body, (b0, g0))
            pltpu.sync_copy(units_v, units_hbm)

        @pl.when(cid == 0)
        def _gather():
            plsc.subcore_barrier()
            base = sid * tpw
            pltpu.sync_copy(units_hbm.at[pl.ds(base, tpw)], idx_v)
            pltpu.async_copy(cb_hbm.at[idx_v], rows_v, sem).wait()
            pltpu.sync_copy(rows_v, quant_hbm.at[pl.ds(base, tpw)])

    return sc_fn


def kernel(features, codebook, lmbda):
    T, _ = features.shape
    K, D = codebook.shape
    lam_arr = jnp.reshape(jnp.asarray(lmbda, jnp.float32), (1,))
    beta2d, gamma2d = _dpdp_beta_gamma(features, codebook, lam_arr)
    sc_fn = _sc_backtrace_gather(T, K, D)
    units, quantized_features = sc_fn(
        jnp.reshape(beta2d, (T,)), jnp.reshape(gamma2d, (T,)), codebook)
    indices = jnp.asarray(units, dtype=jnp.int64)
    quantized_features_st = features - jax.lax.stop_gradient(
        features - quantized_features)
    return (quantized_features_st, indices)


# speedup vs baseline: 1.7795x; 1.7795x over previous
import functools

import jax
import jax.numpy as jnp
from jax import lax
from jax.experimental import pallas as pl
from jax.experimental.pallas import tpu as pltpu
from jax.experimental.pallas import tpu_sc as plsc


def _dpdp_body(lam_ref, f_ref, c_ref, beta_ref, gamma_ref, pc_ref, a_vm,
               gm_ref):
    f = f_ref[:]
    c = c_ref[:]
    T = f.shape[0]
    K = c.shape[0]
    lam = lam_ref[0]

    fn2 = jnp.sum(f * f, axis=1, keepdims=True)
    cn2 = jnp.sum(c * c, axis=1, keepdims=True).T
    g = jax.lax.dot_general(
        f, c, (((1,), (1,)), ((), ())),
        preferred_element_type=jnp.float32,
        precision=jax.lax.Precision.HIGHEST,
    )
    d2 = jnp.maximum(fn2 + cn2 - 2.0 * g, 0.0)

    mu = jnp.sum(d2) / jnp.float32(T * K)
    x = d2 - mu
    sh = 1
    while sh < T:
        x = x + jnp.concatenate([jnp.zeros((sh, K), jnp.float32), x[:-sh]], axis=0)
        sh *= 2
    pc_ref[:] = x

    H = 8
    pc_all = pc_ref[:]
    for i in range(1, H):
        shifted = jnp.concatenate(
            [jnp.zeros((i, K), jnp.float32), pc_all[:-i]], axis=0)
        gcol = jnp.min(pc_all - shifted, axis=1, keepdims=True)
        gm_ref[:, pl.ds(i - 1, 1)] = gcol

    def block_body(b, M):
        s = pl.multiple_of(b * H, H)
        rows = pc_ref[pl.ds(s, H), :]
        vk = jnp.min(rows + M, axis=1, keepdims=True)
        a_blk = []
        for d in range(1, H + 1):
            cand = jnp.min(vk[d - 1:d, :])
            for i in range(1, d):
                g = jnp.min(gm_ref[pl.ds(s + d - 1, 1), pl.ds(i - 1, 1)])
                cand = jnp.minimum(cand, a_blk[d - i - 1] + g)
            a_t = cand + lam
            a_blk.append(a_t)
            a_vm[pl.ds(s + d - 1, 1), :] = jnp.reshape(a_t, (1, 1))
        arows = a_vm[pl.ds(s, H), :]
        return jnp.minimum(M, jnp.min(arows - rows, axis=0, keepdims=True))

    jax.lax.fori_loop(0, T // H, block_body,
                      jnp.zeros((1, K), jnp.float32), unroll=False)

    pc = pc_ref[:]
    avec = a_vm[:]
    qfull = avec - pc
    q = jnp.concatenate([jnp.zeros((1, K), jnp.float32), qfull[:-1]], axis=0)
    mv = q
    ji = jax.lax.broadcasted_iota(jnp.int32, (T, K), 0)
    inf = jnp.float32(jnp.inf)
    sh = 1
    while sh < T:
        pv = jnp.concatenate([jnp.full((sh, K), inf, jnp.float32), mv[:-sh]], axis=0)
        pj = jnp.concatenate([jnp.zeros((sh, K), jnp.int32), ji[:-sh]], axis=0)
        take = pv <= mv
        mv = jnp.where(take, pv, mv)
        ji = jnp.where(take, pj, ji)
        sh *= 2
    r = pc + mv
    rmin = jnp.min(r, axis=1, keepdims=True)
    kiota = jax.lax.broadcasted_iota(jnp.int32, (1, K), 1)
    gam = jnp.min(jnp.where(r == rmin, kiota, K), axis=1, keepdims=True)
    bet = jnp.min(jnp.where(kiota == gam, ji, jnp.int32(2**30)),
                  axis=1, keepdims=True)
    beta_ref[:] = bet
    gamma_ref[:] = gam


def _dpdp_beta_gamma(features, codebook, lam_arr):
    T, _ = features.shape
    K = codebook.shape[0]
    return pl.pallas_call(
        _dpdp_body,
        out_shape=[
            jax.ShapeDtypeStruct((T, 1), jnp.int32),
            jax.ShapeDtypeStruct((T, 1), jnp.int32),
        ],
        in_specs=[
            pl.BlockSpec(memory_space=pltpu.SMEM),
            pl.BlockSpec(memory_space=pltpu.VMEM),
            pl.BlockSpec(memory_space=pltpu.VMEM),
        ],
        out_specs=[
            pl.BlockSpec(memory_space=pltpu.VMEM),
            pl.BlockSpec(memory_space=pltpu.VMEM),
        ],
        scratch_shapes=[
            pltpu.VMEM((T, K), jnp.float32),
            pltpu.VMEM((T, 1), jnp.float32),
            pltpu.VMEM((T, 8), jnp.float32),
        ],
    )(lam_arr, features, codebook)


def _sc_backtrace_gather(T, K, D):
    tpw = T // 16
    mesh = plsc.VectorSubcoreMesh(core_axis_name="c", subcore_axis_name="s")

    @functools.partial(
        pl.kernel,
        mesh=mesh,
        out_type=[
            jax.ShapeDtypeStruct((T,), jnp.int32),
            jax.ShapeDtypeStruct((T, D), jnp.float32),
        ],
        scratch_types=[
            pltpu.VMEM((T + 16,), jnp.int32),
            pltpu.VMEM((T + 16,), jnp.int32),
            pltpu.VMEM((T,), jnp.int32),
            pltpu.VMEM((tpw,), jnp.int32),
            pltpu.VMEM((tpw, D), jnp.float32),
            pltpu.SemaphoreType.DMA,
        ],
    )
    def sc_fn(beta_hbm, gamma_hbm, cb_hbm, units_hbm, quant_hbm,
              beta_v, gamma_v, units_v, idx_v, rows_v, sem):
        cid = lax.axis_index("c")
        sid = lax.axis_index("s")
        liota = lax.broadcasted_iota(jnp.int32, (16,), 0)

        @pl.when((cid == 0) & (sid == 0))
        def _backtrace():
            pltpu.sync_copy(beta_hbm, beta_v.at[pl.ds(0, T)])
            pltpu.sync_copy(gamma_hbm, gamma_v.at[pl.ds(0, T)])

            b0 = beta_v[pl.ds(T - 1, 16)][0]
            g0 = gamma_v[pl.ds(T - 1, 16)][0]

            def pos_body(i, carry):
                b, g = carry
                p = T - 1 - i
                start = pl.multiple_of((p // 16) * 16, 16)
                vec = units_v[pl.ds(start, 16)]
                mask = (start + liota) == p
                units_v[pl.ds(start, 16)] = jnp.where(mask, g, vec)
                hit = p == b
                nxt = jnp.maximum(b - 1, 0)
                nb = beta_v[pl.ds(nxt, 16)][0]
                ng = gamma_v[pl.ds(nxt, 16)][0]
                b = jnp.where(hit, nb, b)
                g = jnp.where(hit, ng, g)
                return (b, g)

            lax.fori_loop(0, T, pos_body, (b0, g0))
            pltpu.sync_copy(units_v, units_hbm)

        @pl.when(cid == 0)
        def _gather():
            plsc.subcore_barrier()
            base = sid * tpw
            pltpu.sync_copy(units_hbm.at[pl.ds(base, tpw)], idx_v)
            pltpu.async_copy(cb_hbm.at[idx_v], rows_v, sem).wait()
            pltpu.sync_copy(rows_v, quant_hbm.at[pl.ds(base, tpw)])

    return sc_fn


def kernel(features, codebook, lmbda):
    T, _ = features.shape
    K, D = codebook.shape
    lam_arr = jnp.reshape(jnp.asarray(lmbda, jnp.float32), (1,))
    beta2d, gamma2d = _dpdp_beta_gamma(features, codebook, lam_arr)
    sc_fn = _sc_backtrace_gather(T, K, D)
    units, quantized_features = sc_fn(
        jnp.reshape(beta2d, (T,)), jnp.reshape(gamma2d, (T,)), codebook)
    indices = jnp.asarray(units, dtype=jnp.int64)
    quantized_features_st = features - jax.lax.stop_gradient(
        features - quantized_features)
    return (quantized_features_st, indices)


# speedup vs baseline: 1.7812x; 1.0010x over previous
import functools

import jax
import jax.numpy as jnp
from jax import lax
from jax.experimental import pallas as pl
from jax.experimental.pallas import tpu as pltpu
from jax.experimental.pallas import tpu_sc as plsc


def _dpdp_body(lam_ref, f_ref, c_ref, beta_ref, gamma_ref, pc_ref, a_vm,
               gm_ref):
    f = f_ref[:]
    c = c_ref[:]
    T = f.shape[0]
    K = c.shape[0]
    lam = lam_ref[0]

    fn2 = jnp.sum(f * f, axis=1, keepdims=True)
    cn2 = jnp.sum(c * c, axis=1, keepdims=True).T
    g = jax.lax.dot_general(
        f, c, (((1,), (1,)), ((), ())),
        preferred_element_type=jnp.float32,
        precision=jax.lax.Precision.HIGHEST,
    )
    d2 = jnp.maximum(fn2 + cn2 - 2.0 * g, 0.0)

    mu = jnp.sum(d2) / jnp.float32(T * K)
    x = d2 - mu
    sh = 1
    while sh < T:
        x = x + jnp.concatenate([jnp.zeros((sh, K), jnp.float32), x[:-sh]], axis=0)
        sh *= 2
    pc_ref[:] = x

    H = 8
    pc_all = pc_ref[:]
    for i in range(1, H):
        shifted = jnp.concatenate(
            [jnp.zeros((i, K), jnp.float32), pc_all[:-i]], axis=0)
        gcol = jnp.min(pc_all - shifted, axis=1, keepdims=True)
        gm_ref[:, pl.ds(i - 1, 1)] = gcol

    def block_body(b, M):
        s = pl.multiple_of(b * H, H)
        rows = pc_ref[pl.ds(s, H), :]
        vk = jnp.min(rows + M, axis=1, keepdims=True)
        gblk = gm_ref[pl.ds(s, H), :]
        a_blk = []
        for d in range(1, H + 1):
            cand = jnp.min(vk[d - 1:d, :])
            for i in range(1, d):
                g = jnp.min(gblk[d - 1:d, i - 1:i])
                cand = jnp.minimum(cand, a_blk[d - i - 1] + g)
            a_t = cand + lam
            a_blk.append(a_t)
            a_vm[pl.ds(s + d - 1, 1), :] = jnp.reshape(a_t, (1, 1))
        arows = a_vm[pl.ds(s, H), :]
        return jnp.minimum(M, jnp.min(arows - rows, axis=0, keepdims=True))

    jax.lax.fori_loop(0, T // H, block_body,
                      jnp.zeros((1, K), jnp.float32), unroll=False)

    pc = pc_ref[:]
    avec = a_vm[:]
    qfull = avec - pc
    q = jnp.concatenate([jnp.zeros((1, K), jnp.float32), qfull[:-1]], axis=0)
    mv = q
    ji = jax.lax.broadcasted_iota(jnp.int32, (T, K), 0)
    inf = jnp.float32(jnp.inf)
    sh = 1
    while sh < T:
        pv = jnp.concatenate([jnp.full((sh, K), inf, jnp.float32), mv[:-sh]], axis=0)
        pj = jnp.concatenate([jnp.zeros((sh, K), jnp.int32), ji[:-sh]], axis=0)
        take = pv <= mv
        mv = jnp.where(take, pv, mv)
        ji = jnp.where(take, pj, ji)
        sh *= 2
    r = pc + mv
    rmin = jnp.min(r, axis=1, keepdims=True)
    kiota = jax.lax.broadcasted_iota(jnp.int32, (1, K), 1)
    gam = jnp.min(jnp.where(r == rmin, kiota, K), axis=1, keepdims=True)
    bet = jnp.min(jnp.where(kiota == gam, ji, jnp.int32(2**30)),
                  axis=1, keepdims=True)
    beta_ref[:] = bet
    gamma_ref[:] = gam


def _dpdp_beta_gamma(features, codebook, lam_arr):
    T, _ = features.shape
    K = codebook.shape[0]
    return pl.pallas_call(
        _dpdp_body,
        out_shape=[
            jax.ShapeDtypeStruct((T, 1), jnp.int32),
            jax.ShapeDtypeStruct((T, 1), jnp.int32),
        ],
        in_specs=[
            pl.BlockSpec(memory_space=pltpu.SMEM),
            pl.BlockSpec(memory_space=pltpu.VMEM),
            pl.BlockSpec(memory_space=pltpu.VMEM),
        ],
        out_specs=[
            pl.BlockSpec(memory_space=pltpu.VMEM),
            pl.BlockSpec(memory_space=pltpu.VMEM),
        ],
        scratch_shapes=[
            pltpu.VMEM((T, K), jnp.float32),
            pltpu.VMEM((T, 1), jnp.float32),
            pltpu.VMEM((T, 8), jnp.float32),
        ],
    )(lam_arr, features, codebook)


def _sc_backtrace_gather(T, K, D):
    tpw = T // 16
    mesh = plsc.VectorSubcoreMesh(core_axis_name="c", subcore_axis_name="s")

    @functools.partial(
        pl.kernel,
        mesh=mesh,
        out_type=[
            jax.ShapeDtypeStruct((T,), jnp.int32),
            jax.ShapeDtypeStruct((T, D), jnp.float32),
        ],
        scratch_types=[
            pltpu.VMEM((T + 16,), jnp.int32),
            pltpu.VMEM((T + 16,), jnp.int32),
            pltpu.VMEM((T,), jnp.int32),
            pltpu.VMEM((tpw,), jnp.int32),
            pltpu.VMEM((tpw, D), jnp.float32),
            pltpu.SemaphoreType.DMA,
        ],
    )
    def sc_fn(beta_hbm, gamma_hbm, cb_hbm, units_hbm, quant_hbm,
              beta_v, gamma_v, units_v, idx_v, rows_v, sem):
        cid = lax.axis_index("c")
        sid = lax.axis_index("s")
        liota = lax.broadcasted_iota(jnp.int32, (16,), 0)

        @pl.when((cid == 0) & (sid == 0))
        def _backtrace():
            pltpu.sync_copy(beta_hbm, beta_v.at[pl.ds(0, T)])
            pltpu.sync_copy(gamma_hbm, gamma_v.at[pl.ds(0, T)])

            b0 = beta_v[pl.ds(T - 1, 16)][0]
            g0 = gamma_v[pl.ds(T - 1, 16)][0]

            def pos_body(i, carry):
                b, g = carry
                p = T - 1 - i
                start = pl.multiple_of((p // 16) * 16, 16)
                vec = units_v[pl.ds(start, 16)]
                mask = (start + liota) == p
                units_v[pl.ds(start, 16)] = jnp.where(mask, g, vec)
                hit = p == b
                nxt = jnp.maximum(b - 1, 0)
                nb = beta_v[pl.ds(nxt, 16)][0]
                ng = gamma_v[pl.ds(nxt, 16)][0]
                b = jnp.where(hit, nb, b)
                g = jnp.where(hit, ng, g)
                return (b, g)

            lax.fori_loop(0, T, pos_body, (b0, g0))
            pltpu.sync_copy(units_v, units_hbm)

        @pl.when(cid == 0)
        def _gather():
            plsc.subcore_barrier()
            base = sid * tpw
            pltpu.sync_copy(units_hbm.at[pl.ds(base, tpw)], idx_v)
            pltpu.async_copy(cb_hbm.at[idx_v], rows_v, sem).wait()
            pltpu.sync_copy(rows_v, quant_hbm.at[pl.ds(base, tpw)])

    return sc_fn


def kernel(features, codebook, lmbda):
    T, _ = features.shape
    K, D = codebook.shape
    lam_arr = jnp.reshape(jnp.asarray(lmbda, jnp.float32), (1,))
    beta2d, gamma2d = _dpdp_beta_gamma(features, codebook, lam_arr)
    sc_fn = _sc_backtrace_gather(T, K, D)
    units, quantized_features = sc_fn(
        jnp.reshape(beta2d, (T,)), jnp.reshape(gamma2d, (T,)), codebook)
    indices = jnp.asarray(units, dtype=jnp.int64)
    quantized_features_st = features - jax.lax.stop_gradient(
        features - quantized_features)
    return (quantized_features_st, indices)
